# baseline (speedup 1.0000x reference)
import functools
import jax
import jax.numpy as jnp
from jax import lax
from jax.experimental import pallas as pl
from jax.experimental.pallas import tpu as pltpu
from jax.experimental.pallas import tpu_sc as plsc

FEATS_ = 128
K_ = 50000
CHUNK_ = 80
NCHUNKS_ = K_ // CHUNK_
NW_ = 32
NGROUP_ = CHUNK_ // 16


_GDN = lax.GatherDimensionNumbers(
    offset_dims=(), collapsed_slice_dims=(0,), start_index_map=(0,))


def _shuffle(v, idx):
    return lax.gather(v, idx[:, None], _GDN, (1,),
                      mode=lax.GatherScatterMode.PROMISE_IN_BOUNDS)


def _lanesum(v):
    idx = lax.iota(jnp.int32, 16)
    for k in (8, 4, 2, 1):
        v = v + _shuffle(v, idx ^ k)
    return v


def _tanh16(s):
    e = jnp.exp(-2.0 * jnp.abs(s))
    t = (1.0 - e) / (1.0 + e)
    return jnp.sign(s) * t


def _sc_kernel(x_hbm, w_hbm, o_hbm, buf, wv, insem, outsem):
    wid = lax.axis_index("s") * 2 + lax.axis_index("c")
    nw = jnp.where(wid < 17, 20, 19)

    pltpu.sync_copy(w_hbm, wv)
    wregs = [wv[pl.ds(16 * d, 16)] for d in range(8)]

    def chunk_row0(i):
        return (wid + NW_ * i) * CHUNK_

    def in_copy(i, slot):
        return pltpu.make_async_copy(
            x_hbm.at[pl.ds(chunk_row0(i), CHUNK_), :], buf.at[slot],
            insem.at[slot])

    def out_copy(i, slot):
        return pltpu.make_async_copy(
            buf.at[slot], o_hbm.at[pl.ds(chunk_row0(i), CHUNK_), :],
            outsem.at[slot])

    in_copy(0, 0).start()

    def body(i, _):
        slot = lax.rem(i, 2)
        nslot = 1 - slot

        @pl.when(i >= 1)
        def _():
            out_copy(i - 1, nslot).wait()

        @pl.when(i + 1 < nw)
        def _():
            in_copy(i + 1, nslot).start()

        in_copy(i, slot).wait()

        def grp(g, _):
            for l in range(16):
                row = g * 16 + l
                xr = [buf[slot, row, pl.ds(16 * d, 16)] for d in range(8)]
                acc = xr[0] * wregs[0]
                for d in range(1, 8):
                    acc = acc + xr[d] * wregs[d]
                t = _tanh16(_lanesum(acc))
                for d in range(8):
                    buf[slot, row, pl.ds(16 * d, 16)] = xr[d] * t
            return 0

        lax.fori_loop(0, NGROUP_, grp, 0)
        out_copy(i, slot).start()
        return 0

    lax.fori_loop(0, nw, body, 0)
    out_copy(nw - 1, lax.rem(nw - 1, 2)).wait()


def kernel(node_embs, mask, scorer):
    del mask
    run = functools.partial(
        pl.kernel,
        out_type=jax.ShapeDtypeStruct((K_, FEATS_), jnp.float32),
        mesh=plsc.VectorSubcoreMesh(core_axis_name="c", subcore_axis_name="s"),
        scratch_types=[
            pltpu.VMEM((2, CHUNK_, FEATS_), jnp.float32),
            pltpu.VMEM((FEATS_,), jnp.float32),
            pltpu.SemaphoreType.DMA((2,)),
            pltpu.SemaphoreType.DMA((2,)),
        ],
    )(_sc_kernel)
    w_unit = scorer.reshape(FEATS_) / jnp.linalg.norm(scorer)
    out = run(node_embs, w_unit)
    return out.T


# speedup vs baseline: 5.0136x; 5.0136x over previous
import jax
import jax.numpy as jnp
from jax import lax
from jax.experimental import pallas as pl

FEATS_ = 128
K_ = 50000
BLOCK_ = 16672


def _scale_kernel(x_ref, w_ref, o_ref):
    x = x_ref[...]
    w = w_ref[...]
    inv_norm = jax.lax.rsqrt(jnp.sum(w * w))
    s = jnp.dot(x, w, preferred_element_type=jnp.float32) * inv_norm
    o_ref[...] = x * jnp.tanh(s)


def kernel(node_embs, mask, scorer):
    del mask
    n_blocks = pl.cdiv(K_, BLOCK_)
    out = pl.pallas_call(
        _scale_kernel,
        grid=(n_blocks,),
        in_specs=[
            pl.BlockSpec((BLOCK_, FEATS_), lambda i: (i, 0)),
            pl.BlockSpec((FEATS_, 1), lambda i: (0, 0)),
        ],
        out_specs=pl.BlockSpec((BLOCK_, FEATS_), lambda i: (i, 0)),
        out_shape=jax.ShapeDtypeStruct((K_, FEATS_), jnp.float32),
    )(node_embs, scorer)
    return out.T
